# baseline (speedup 1.0000x reference)
import functools

import jax
import jax.numpy as jnp
from jax import lax
from jax.experimental import pallas as pl
from jax.experimental.pallas import tpu as pltpu
from jax.experimental.pallas import tpu_sc as plsc

_P_SPATIAL = 21.0
_T_STEP = 3600.0
_X_STORAGE = 0.29
_SLOPE_MIN = 0.0001
_SLOPE_MAX = 0.3

_L = 5
_PLV = 10240
_NP = _L * _PLV
_NT = 16
_W = _PLV // _NT
_NGRP = _W // 16


_LN_COEF = (0.010118902, -0.12345627, 0.65900403, -2.0201724,
            3.9325855, -5.1266217, 4.911019, -2.2424765)


def _vlog(x):
    bits = plsc.bitcast(x, jnp.int32)
    e = (bits >> 23) - 127
    m = plsc.bitcast((bits & 0x007FFFFF) | 0x3F800000, jnp.float32)
    acc = jnp.full((16,), _LN_COEF[0], jnp.float32)
    for c in _LN_COEF[1:]:
        acc = acc * m + c
    return e.astype(jnp.float32) * 0.6931472 + acc


def _routing_kernel(T):
    mesh = plsc.VectorSubcoreMesh(core_axis_name="c", subcore_axis_name="s")

    @functools.partial(
        pl.kernel,
        mesh=mesh,
        compiler_params=pltpu.CompilerParams(needs_layout_passes=False),
        out_type=jax.ShapeDtypeStruct((T * 16,), jnp.float32),
        scratch_types=[
            pltpu.VMEM((_NP + 16,), jnp.float32),
            pltpu.VMEM((_NP + 16,), jnp.float32),
            pltpu.VMEM((_W,), jnp.float32),
            pltpu.VMEM((_L * _W,), jnp.int32),
            pltpu.VMEM((_L * _W,), jnp.int32),
            pltpu.VMEM((_L * _W,), jnp.int32),
            pltpu.VMEM((_L * _W,), jnp.float32),
            pltpu.VMEM((_L * _W,), jnp.float32),
            pltpu.VMEM((_L * _W,), jnp.float32),
            pltpu.VMEM((_L * _W,), jnp.float32),
            pltpu.VMEM((64,), jnp.int32),
            pltpu.VMEM((16,), jnp.float32),
            pltpu.VMEM((T * 16,), jnp.float32),
            pltpu.VMEM((_W,), jnp.float32),
            pltpu.VMEM((_W,), jnp.float32),
            pltpu.VMEM((_W,), jnp.float32),
            pltpu.VMEM((_W,), jnp.float32),
            pltpu.VMEM_SHARED((2 * _PLV,), jnp.float32),
            pltpu.SemaphoreType.DMA,
        ],
    )
    def k(b_hbm, la_hbm, lb_hbm, qp_hbm, q0_hbm, at_hbm, gi_hbm, p_hbm, out_hbm,
          dX, dY, qt1, a0, a1, a2, ball, laall, lball, qpb, gbuf, pbuf, obuf,
          sden, sc1, sn1, sS, spbuf, psem):
        cid = lax.axis_index("c")
        sid = lax.axis_index("s")

        def body():
            w0 = sid * _W
            pltpu.sync_copy(p_hbm, pbuf)
            pltpu.sync_copy(q0_hbm, dX.at[pl.ds(0, _NP)])
            dX[pl.ds(_NP, 16)] = jnp.zeros((16,), jnp.float32)
            dY[pl.ds(_NP, 16)] = jnp.zeros((16,), jnp.float32)
            for lvl in range(_L):
                src = lvl * _PLV + w0
                dst = lvl * _W
                pltpu.sync_copy(at_hbm.at[pl.ds(src, _W)], a0.at[pl.ds(dst, _W)])
                pltpu.sync_copy(at_hbm.at[pl.ds(_NP + src, _W)], a1.at[pl.ds(dst, _W)])
                pltpu.sync_copy(at_hbm.at[pl.ds(2 * _NP + src, _W)], a2.at[pl.ds(dst, _W)])
                pltpu.sync_copy(b_hbm.at[pl.ds(src, _W)], ball.at[pl.ds(dst, _W)])
                pltpu.sync_copy(la_hbm.at[pl.ds(src, _W)], laall.at[pl.ds(dst, _W)])
                pltpu.sync_copy(lb_hbm.at[pl.ds(src, _W)], lball.at[pl.ds(dst, _W)])
            p_v = pbuf[...]
            lane = lax.iota(jnp.int32, 16)

            pl.when(sid == 0)(lambda: pltpu.sync_copy(gi_hbm, gbuf))

            def readout(d_cur, ts):
                row = jnp.zeros((16,), jnp.float32)
                for g in range(4):
                    idxv = gbuf[pl.ds(g * 16, 16)]
                    vals = plsc.load_gather(d_cur, [idxv])
                    vals = jnp.where(lane < 8, vals, 0.0)
                    s = jnp.sum(vals)
                    row = jnp.where(lane == g, s, row)
                obuf[pl.ds(ts * 16, 16)] = row

            pl.when(sid == 0)(lambda: readout(dX, 0))

            bufs = [dX, dY]
            for ts in range(1, T):
                d_old = bufs[(ts + 1) % 2]
                d_new = bufs[ts % 2]
                pltpu.sync_copy(
                    qp_hbm.at[pl.ds(ts * _NP + sid * (_L * _W), _L * _W)], qpb)

                def level_body(lvl, carry, d_old=d_old, d_new=d_new, ts=ts):
                    base = lvl * _PLV
                    goff = base + w0
                    loff = lvl * _W
                    soff = ((lvl + ts) & 1) * _PLV
                    psoff = ((lvl + ts + 1) & 1) * _PLV

                    pbase = jnp.maximum(lvl - 1, 0) * _PLV
                    pull = pltpu.async_copy(
                        spbuf.at[pl.ds(psoff, _PLV)],
                        d_new.at[pl.ds(pbase, _PLV)], psem)

                    def pass1_group(s):
                        q_t = d_old[pl.ds(goff + s, 16)]
                        i_t = jnp.zeros((16,), jnp.float32)
                        cold = jnp.zeros((16,), jnp.float32)
                        for abuf in (a0, a1, a2):
                            idx = abuf[pl.ds(loff + s, 16)]
                            go = plsc.load_gather(d_old, [idx])
                            i_t = i_t + go
                            cold = cold + jnp.where(idx < base, go, 0.0)
                        v = ball[pl.ds(loff + s, 16)] * jnp.exp(p_v * _vlog(q_t))
                        cv = jnp.minimum(jnp.maximum(v, 0.3), 15.0)
                        u = _T_STEP * cv
                        lb_v = lball[pl.ds(loff + s, 16)]
                        w2 = qpb[pl.ds(loff + s, 16)] - q_t
                        sden[pl.ds(s, 16)] = laall[pl.ds(loff + s, 16)] + u
                        sc1[pl.ds(s, 16)] = u - lb_v
                        sn1[pl.ds(s, 16)] = (w2 + w2) * u + (u + lb_v) * i_t
                        if ts == 1:
                            sS[pl.ds(s, 16)] = jnp.zeros((16,), jnp.float32)
                        else:
                            sS[pl.ds(s, 16)] = i_t - cold

                    def pass2_group(s):
                        cnew = jnp.zeros((16,), jnp.float32)
                        for abuf in (a0, a1, a2):
                            idx = abuf[pl.ds(loff + s, 16)]
                            gn = plsc.load_gather(d_new, [idx])
                            cnew = cnew + jnp.where(idx < base, gn, 0.0)
                        i_t1 = cnew + sS[pl.ds(s, 16)]
                        numer = sn1[pl.ds(s, 16)] + sc1[pl.ds(s, 16)] * i_t1
                        q1 = numer / sden[pl.ds(s, 16)] + d_old[pl.ds(goff + s, 16)]
                        qt1[pl.ds(s, 16)] = jnp.maximum(q1, 0.0001)

                    def p1_body(i, carry3):
                        pass1_group(i * 32)
                        pass1_group(i * 32 + 16)
                        return carry3

                    def p2_body(i, carry3):
                        pass2_group(i * 32)
                        pass2_group(i * 32 + 16)
                        return carry3

                    lax.fori_loop(0, _NGRP // 2, p1_body, 0)
                    pull.wait()
                    lax.fori_loop(0, _NGRP // 2, p2_body, 0)
                    pltpu.sync_copy(qt1, spbuf.at[pl.ds(soff + w0, _W)])
                    plsc.subcore_barrier()
                    return carry

                lax.fori_loop(0, _L, level_body, 0)
                last_soff = ((_L - 1 + ts) & 1) * _PLV
                pltpu.sync_copy(spbuf.at[pl.ds(last_soff, _PLV)],
                                d_new.at[pl.ds((_L - 1) * _PLV, _PLV)])
                pl.when(sid == 0)(lambda d_new=d_new, ts=ts: readout(d_new, ts))

            pl.when(sid == 0)(lambda: pltpu.sync_copy(obuf, out_hbm))

        pl.when(cid == 0)(body)

    return k


def kernel(attributes, q_prime, n_param, q_spatial_param, river_index_graph, A, gage_indices):
    T, N = q_prime.shape
    NL = N // _L

    slope = jnp.clip(attributes[:, 1], _SLOPE_MIN, _SLOPE_MAX)
    ss = jnp.sqrt(slope)
    p = 2.0 / (5.0 + 3.0 * q_spatial_param)
    a = n_param * (q_spatial_param + 1.0) / (_P_SPATIAL * ss)
    b = (1.0 / n_param) * ss * jnp.power(a, p)
    la = (1.42 * 0.6) * attributes[:, 0]
    lb = (0.58 * 0.6) * attributes[:, 0]

    def padv(x, fill):
        x2 = x.reshape(_L, NL)
        return jnp.pad(x2, ((0, 0), (0, _PLV - NL)), constant_values=fill).reshape(_NP)

    bp = padv(b.astype(jnp.float32), 1.0)
    lap = padv(la.astype(jnp.float32), 1.0)
    lbp = padv(lb.astype(jnp.float32), 1.0)
    qpad = jax.vmap(lambda r: padv(r, 1.0))(q_prime)
    q0p = qpad[0]
    qpp = (qpad.reshape(T, _L, _NT, _W).transpose(0, 2, 1, 3).reshape(T * _NP))

    pos = lambda x: (x // NL) * _PLV + (x % NL)
    Ar = jnp.where(A >= 0, pos(A), _NP).astype(jnp.int32)
    Arp = jnp.full((_L, _PLV, 3), _NP, dtype=jnp.int32)
    Arp = Arp.at[:, :NL, :].set(Ar.reshape(_L, NL, 3))
    Arp = Arp.at[0].set(_NP)
    atp = Arp.transpose(2, 0, 1).reshape(3 * _NP)

    gp = pos(gage_indices).astype(jnp.int32)
    gip = jnp.pad(gp, ((0, 0), (0, 8))).reshape(64)
    pvec = jnp.full((16,), p, dtype=jnp.float32)

    out = _routing_kernel(T)(bp, lap, lbp, qpp, q0p, atp, gip, pvec)
    return out.reshape(T, 16)[:, :4].T


# speedup vs baseline: 1.2004x; 1.2004x over previous
import functools

import jax
import jax.numpy as jnp
from jax import lax
from jax.experimental import pallas as pl
from jax.experimental.pallas import tpu as pltpu
from jax.experimental.pallas import tpu_sc as plsc

_P_SPATIAL = 21.0
_T_STEP = 3600.0
_X_STORAGE = 0.29
_SLOPE_MIN = 0.0001
_SLOPE_MAX = 0.3

_L = 5
_PLV = 10240
_NP = _L * _PLV
_NT = 16
_W = _PLV // _NT
_NGRP = _W // 16


_LN_COEF = (0.010118902, -0.12345627, 0.65900403, -2.0201724,
            3.9325855, -5.1266217, 4.911019, -2.2424765)


def _vlog(x):
    bits = plsc.bitcast(x, jnp.int32)
    e = (bits >> 23) - 127
    m = plsc.bitcast((bits & 0x007FFFFF) | 0x3F800000, jnp.float32)
    acc = jnp.full((16,), _LN_COEF[0], jnp.float32)
    for c in _LN_COEF[1:]:
        acc = acc * m + c
    return e.astype(jnp.float32) * 0.6931472 + acc


def _routing_kernel(T):
    mesh = plsc.VectorSubcoreMesh(core_axis_name="c", subcore_axis_name="s")

    @functools.partial(
        pl.kernel,
        mesh=mesh,
        compiler_params=pltpu.CompilerParams(needs_layout_passes=False),
        out_type=jax.ShapeDtypeStruct((T * 16,), jnp.float32),
        scratch_types=[
            pltpu.VMEM((_NP + 16,), jnp.float32),
            pltpu.VMEM((_NP + 16,), jnp.float32),
            pltpu.VMEM((_W,), jnp.float32),
            pltpu.VMEM((_L * _W,), jnp.int32),
            pltpu.VMEM((_L * _W,), jnp.int32),
            pltpu.VMEM((_L * _W,), jnp.int32),
            pltpu.VMEM((_L * _W,), jnp.float32),
            pltpu.VMEM((_L * _W,), jnp.float32),
            pltpu.VMEM((_L * _W,), jnp.float32),
            pltpu.VMEM((2 * _L * _W,), jnp.float32),
            pltpu.VMEM((64,), jnp.int32),
            pltpu.VMEM((16,), jnp.float32),
            pltpu.VMEM((T * 16,), jnp.float32),
            pltpu.VMEM_SHARED((2 * _PLV,), jnp.float32),
            pltpu.SemaphoreType.DMA,
            pltpu.SemaphoreType.DMA,
        ],
    )
    def k(b_hbm, la_hbm, lb_hbm, qp_hbm, q0_hbm, at_hbm, gi_hbm, p_hbm, out_hbm,
          dX, dY, qt1, a0, a1, a2, ball, laall, lball, qpb, gbuf, pbuf, obuf,
          spbuf, isem, qsem):
        cid = lax.axis_index("c")
        sid = lax.axis_index("s")

        def body():
            w0 = sid * _W
            pend = [pltpu.async_copy(p_hbm, pbuf, isem),
                    pltpu.async_copy(q0_hbm, dX.at[pl.ds(0, _NP)], isem)]
            for lvl in range(_L):
                src = lvl * _PLV + w0
                dst = lvl * _W
                pend += [
                    pltpu.async_copy(at_hbm.at[pl.ds(src, _W)],
                                     a0.at[pl.ds(dst, _W)], isem),
                    pltpu.async_copy(at_hbm.at[pl.ds(_NP + src, _W)],
                                     a1.at[pl.ds(dst, _W)], isem),
                    pltpu.async_copy(at_hbm.at[pl.ds(2 * _NP + src, _W)],
                                     a2.at[pl.ds(dst, _W)], isem),
                    pltpu.async_copy(b_hbm.at[pl.ds(src, _W)],
                                     ball.at[pl.ds(dst, _W)], isem),
                    pltpu.async_copy(la_hbm.at[pl.ds(src, _W)],
                                     laall.at[pl.ds(dst, _W)], isem),
                    pltpu.async_copy(lb_hbm.at[pl.ds(src, _W)],
                                     lball.at[pl.ds(dst, _W)], isem),
                ]
            qnext = pltpu.async_copy(
                qp_hbm.at[pl.ds(1 * _NP + sid * (_L * _W), _L * _W)],
                qpb.at[pl.ds(0, _L * _W)], qsem)
            dX[pl.ds(_NP, 16)] = jnp.zeros((16,), jnp.float32)
            dY[pl.ds(_NP, 16)] = jnp.zeros((16,), jnp.float32)
            for h in pend:
                h.wait()
            p_v = pbuf[...]
            lane = lax.iota(jnp.int32, 16)

            pl.when(sid == 0)(lambda: pltpu.sync_copy(gi_hbm, gbuf))

            def readout(d_cur, ts):
                row = jnp.zeros((16,), jnp.float32)
                for g in range(4):
                    idxv = gbuf[pl.ds(g * 16, 16)]
                    vals = plsc.load_gather(d_cur, [idxv])
                    vals = jnp.where(lane < 8, vals, 0.0)
                    s = jnp.sum(vals)
                    row = jnp.where(lane == g, s, row)
                obuf[pl.ds(ts * 16, 16)] = row

            pl.when(sid == 0)(lambda: readout(dX, 0))

            bufs = [dX, dY]
            for ts in range(1, T):
                d_old = bufs[(ts + 1) % 2]
                d_new = bufs[ts % 2]
                qoff = ((ts - 1) & 1) * (_L * _W)
                qnext.wait()
                if ts + 1 < T:
                    qnext = pltpu.async_copy(
                        qp_hbm.at[pl.ds((ts + 1) * _NP + sid * (_L * _W), _L * _W)],
                        qpb.at[pl.ds((ts & 1) * (_L * _W), _L * _W)], qsem)

                def level_body(lvl, carry, d_old=d_old, d_new=d_new, ts=ts, qoff=qoff):
                    base = lvl * _PLV
                    goff = base + w0
                    loff = lvl * _W
                    soff = ((lvl + ts) & 1) * _PLV

                    def compute_group(s):
                        q_t = d_old[pl.ds(goff + s, 16)]
                        i_t = jnp.zeros((16,), jnp.float32)
                        i_t1 = jnp.zeros((16,), jnp.float32)
                        for abuf in (a0, a1, a2):
                            idx = abuf[pl.ds(loff + s, 16)]
                            go = plsc.load_gather(d_old, [idx])
                            gn = plsc.load_gather(d_new, [idx])
                            i_t = i_t + go
                            prev = jnp.zeros((16,), jnp.float32) if ts == 1 else go
                            i_t1 = i_t1 + jnp.where(idx < base, gn, prev)
                        v = ball[pl.ds(loff + s, 16)] * jnp.exp(p_v * _vlog(q_t))
                        cv = jnp.minimum(jnp.maximum(v, 0.3), 15.0)
                        u = _T_STEP * cv
                        den = laall[pl.ds(loff + s, 16)] + u
                        w2 = qpb[pl.ds(qoff + loff + s, 16)] - q_t
                        numer = u * (i_t1 + i_t + (w2 + w2)) \
                            + lball[pl.ds(loff + s, 16)] * (i_t - i_t1)
                        q1 = numer / den + q_t
                        qt1[pl.ds(s, 16)] = jnp.maximum(q1, 0.0001)

                    def grp_body(i, carry3):
                        compute_group(i * 32)
                        compute_group(i * 32 + 16)
                        return carry3

                    lax.fori_loop(0, _NGRP // 2, grp_body, 0)
                    pltpu.sync_copy(qt1, spbuf.at[pl.ds(soff + w0, _W)])
                    plsc.subcore_barrier()
                    pltpu.sync_copy(spbuf.at[pl.ds(soff, _PLV)], d_new.at[pl.ds(base, _PLV)])
                    return carry

                lax.fori_loop(0, _L, level_body, 0)
                pl.when(sid == 0)(lambda d_new=d_new, ts=ts: readout(d_new, ts))

            pl.when(sid == 0)(lambda: pltpu.sync_copy(obuf, out_hbm))

        pl.when(cid == 0)(body)

    return k


def kernel(attributes, q_prime, n_param, q_spatial_param, river_index_graph, A, gage_indices):
    T, N = q_prime.shape
    NL = N // _L

    slope = jnp.clip(attributes[:, 1], _SLOPE_MIN, _SLOPE_MAX)
    ss = jnp.sqrt(slope)
    p = 2.0 / (5.0 + 3.0 * q_spatial_param)
    a = n_param * (q_spatial_param + 1.0) / (_P_SPATIAL * ss)
    b = (1.0 / n_param) * ss * jnp.power(a, p)
    la = (1.42 * 0.6) * attributes[:, 0]
    lb = (0.58 * 0.6) * attributes[:, 0]

    def padv(x, fill):
        x2 = x.reshape(_L, NL)
        return jnp.pad(x2, ((0, 0), (0, _PLV - NL)), constant_values=fill).reshape(_NP)

    bp = padv(b.astype(jnp.float32), 1.0)
    lap = padv(la.astype(jnp.float32), 1.0)
    lbp = padv(lb.astype(jnp.float32), 1.0)
    qpad = jax.vmap(lambda r: padv(r, 1.0))(q_prime)
    q0p = qpad[0]
    qpp = (qpad.reshape(T, _L, _NT, _W).transpose(0, 2, 1, 3).reshape(T * _NP))

    pos = lambda x: (x // NL) * _PLV + (x % NL)
    Ar = jnp.where(A >= 0, pos(A), _NP).astype(jnp.int32)
    Arp = jnp.full((_L, _PLV, 3), _NP, dtype=jnp.int32)
    Arp = Arp.at[:, :NL, :].set(Ar.reshape(_L, NL, 3))
    Arp = Arp.at[0].set(_NP)
    atp = Arp.transpose(2, 0, 1).reshape(3 * _NP)

    gp = pos(gage_indices).astype(jnp.int32)
    gip = jnp.pad(gp, ((0, 0), (0, 8))).reshape(64)
    pvec = jnp.full((16,), p, dtype=jnp.float32)

    out = _routing_kernel(T)(bp, lap, lbp, qpp, q0p, atp, gip, pvec)
    return out.reshape(T, 16)[:, :4].T
